# baseline (speedup 1.0000x reference)
import jax
import jax.numpy as jnp
from jax import lax
from jax.experimental import pallas as pl
from jax.experimental.pallas import tpu as pltpu
from jax.experimental.pallas import tpu_sc as plsc

T, N, F, H, E = 8, 10000, 256, 128, 160000
NC, NS = 2, 16
K = 128
EPT = E // NS
EPAD = 10240
CH = EPAD // K
HALF = N // NC
ROFF2 = 312
RSZ2 = 320
TRB = 5008
NTR = 128
SROWS = TRB + NTR

_mesh = plsc.VectorSubcoreMesh(core_axis_name="c", subcore_axis_name="s",
                               num_cores=NC, num_subcores=NS)


def _mk_local_dst(c, dst_v, lane):
    def mkldst(i, carry):
        for q in range(K // 16):
            d = dst_v[i, pl.ds(q * 16, 16)] - c * HALF
            owned = (d >= 0) & (d < HALF)
            trash = TRB + ((q * 16 + lane) & (NTR - 1))
            dst_v[i, pl.ds(q * 16, 16)] = jnp.where(owned, d, trash)
        return carry

    lax.fori_loop(0, CH, mkldst, 0)


def _deg_body(dstp, ones_h, zer_h, deg_out, dst_v, ones_v, zv, spm):
    c = lax.axis_index("c")
    s = lax.axis_index("s")
    pltpu.sync_copy(dstp.at[s], dst_v)
    pltpu.sync_copy(ones_h, ones_v)
    pltpu.sync_copy(zer_h, zv)
    lane = lax.iota(jnp.int32, 16)
    _mk_local_dst(c, dst_v, lane)
    pltpu.sync_copy(zv, spm.at[pl.ds(s * ROFF2, RSZ2)])
    plsc.subcore_barrier()

    def chunk(j, carry):
        pltpu.sync_copy(ones_v, spm.at[dst_v.at[j]], add=True)
        return carry

    lax.fori_loop(0, CH, chunk, 0)
    plsc.subcore_barrier()
    pltpu.sync_copy(spm.at[pl.ds(s * ROFF2, RSZ2)], zv)
    pltpu.sync_copy(zv, deg_out.at[pl.ds(c * HALF + s * ROFF2, RSZ2)])


_deg_kernel = pl.kernel(
    _deg_body,
    out_type=jax.ShapeDtypeStruct((N, H), jnp.float32),
    mesh=_mesh,
    scratch_types=[
        pltpu.VMEM((CH, K), jnp.int32),
        pltpu.VMEM((K, H), jnp.float32),
        pltpu.VMEM((RSZ2, H), jnp.float32),
        pltpu.VMEM_SHARED((SROWS, H), jnp.float32),
    ],
)


BN_B = 1000


def _gcn_mm_body(x_ref, w_ref, deg_ref, out_ref):
    dis = lax.rsqrt(deg_ref[:, 0] + 1.0)
    h = jnp.dot(x_ref[0], w_ref[...], preferred_element_type=jnp.float32)
    out_ref[0] = h * dis[:, None]


def _gcn_mm(X_seq, W_g, deg):
    return pl.pallas_call(
        _gcn_mm_body,
        grid=(N // BN_B, T),
        in_specs=[
            pl.BlockSpec((1, BN_B, F), lambda n, t: (t, n, 0)),
            pl.BlockSpec((F, H), lambda n, t: (0, 0)),
            pl.BlockSpec((BN_B, H), lambda n, t: (n, 0)),
        ],
        out_specs=pl.BlockSpec((1, BN_B, H), lambda n, t: (t, n, 0)),
        out_shape=jax.ShapeDtypeStruct((T, N, H), jnp.float32),
    )(X_seq, W_g, deg)


TH = T // NC
EPT2 = E // NS
K2 = 64
EPAD2 = 10240
NHLV = 2
CH2 = EPAD2 // K2 // NHLV
ROFF = 624
RSZ = 640
NB = RSZ // K2
AROWS = N + 8


def _agg_body(hs, srcp, dstp, vout, src_v, dst_v, rows0, rows1,
              spm, semg0, semg1, sems0, sems1):
    c = lax.axis_index("c")
    s = lax.axis_index("s")
    zv = jnp.zeros((16,), jnp.float32)

    def zrow(i, carry):
        for q in range(H // 16):
            rows0[i, pl.ds(q * 16, 16)] = zv
        return carry

    for t in range(TH):
        lax.fori_loop(0, K2, zrow, 0)
        for p in range(NB):
            pltpu.sync_copy(rows0, spm.at[pl.ds(s * ROFF + p * K2, K2)])
        plsc.subcore_barrier()
        off = c * (TH * N) + t * N
        for half in range(NHLV):
            pltpu.sync_copy(srcp.at[s, half], src_v)
            pltpu.sync_copy(dstp.at[s, half], dst_v)

            def bump(i, carry):
                for q in range(K2 // 16):
                    src_v[i, pl.ds(q * 16, 16)] = (
                        src_v[i, pl.ds(q * 16, 16)] + off)
                return carry

            lax.fori_loop(0, CH2, bump, 0)
            pltpu.async_copy(hs.at[src_v.at[0]], rows0, semg0)
            pltpu.async_copy(hs.at[src_v.at[1]], rows1, semg1)

            def pair(m, carry):
                j0 = 2 * m
                pltpu.make_async_copy(hs.at[src_v.at[j0]], rows0,
                                      semg0).wait()
                pltpu.async_copy(rows0, spm.at[dst_v.at[j0]], sems0,
                                 add=True)

                pltpu.make_async_copy(hs.at[src_v.at[j0 + 1]], rows1,
                                      semg1).wait()
                pltpu.async_copy(rows1, spm.at[dst_v.at[j0 + 1]], sems1,
                                 add=True)

                @pl.when(m < CH2 // 2 - 1)
                def _():
                    pltpu.make_async_copy(rows0, spm.at[dst_v.at[j0]],
                                          sems0).wait()
                    pltpu.async_copy(hs.at[src_v.at[j0 + 2]], rows0, semg0)
                    pltpu.make_async_copy(rows1, spm.at[dst_v.at[j0 + 1]],
                                          sems1).wait()
                    pltpu.async_copy(hs.at[src_v.at[j0 + 3]], rows1, semg1)

                return carry

            lax.fori_loop(0, CH2 // 2, pair, 0)
            pltpu.make_async_copy(rows0, spm.at[dst_v.at[CH2 - 2]],
                                  sems0).wait()
            pltpu.make_async_copy(rows1, spm.at[dst_v.at[CH2 - 1]],
                                  sems1).wait()
        plsc.subcore_barrier()
        base = (c * TH + t) * N
        for p in range(NB):
            buf, sem = (rows0, semg0) if p % 2 == 0 else (rows1, semg1)
            if p >= 2:
                pltpu.make_async_copy(
                    buf,
                    vout.at[pl.ds(base + s * ROFF + (p - 2) * K2, K2)],
                    sem).wait()
            pltpu.sync_copy(spm.at[pl.ds(s * ROFF + p * K2, K2)], buf)
            pltpu.async_copy(
                buf, vout.at[pl.ds(base + s * ROFF + p * K2, K2)], sem)
        for p in (NB - 2, NB - 1):
            buf, sem = (rows0, semg0) if p % 2 == 0 else (rows1, semg1)
            pltpu.make_async_copy(
                buf, vout.at[pl.ds(base + s * ROFF + p * K2, K2)],
                sem).wait()


_agg_kernel = pl.kernel(
    _agg_body,
    out_type=jax.ShapeDtypeStruct((T * N, H), jnp.float32),
    mesh=_mesh,
    scratch_types=[
        pltpu.VMEM((CH2, K2), jnp.int32),
        pltpu.VMEM((CH2, K2), jnp.int32),
        pltpu.VMEM((K2, H), jnp.float32),
        pltpu.VMEM((K2, H), jnp.float32),
        pltpu.VMEM_SHARED((AROWS, H), jnp.float32),
        pltpu.SemaphoreType.DMA,
        pltpu.SemaphoreType.DMA,
        pltpu.SemaphoreType.DMA,
        pltpu.SemaphoreType.DMA,
    ],
)


BN_D = 1000


def _lstm_body(v_ref, hs_ref, deg_ref, wih_ref, whh_ref, bih_ref, bhh_ref,
               bg_ref, wfc_ref, bfc_ref, out_ref):
    dis = lax.rsqrt(deg_ref[:, 0] + 1.0)[:, None]
    bias = bih_ref[0] + bhh_ref[0]
    bg = bg_ref[0]
    h = jnp.zeros((BN_D, H), jnp.float32)
    cc = jnp.zeros((BN_D, H), jnp.float32)
    for t in range(T):
        v_t = v_ref[t] + hs_ref[t]
        g_in = v_t * dis + bg
        gates = (jnp.dot(g_in, wih_ref[...],
                         preferred_element_type=jnp.float32)
                 + jnp.dot(h, whh_ref[...],
                           preferred_element_type=jnp.float32)
                 + bias)
        ig = jax.nn.sigmoid(gates[:, 0:H])
        fg = jax.nn.sigmoid(gates[:, H:2 * H])
        gg = jnp.tanh(gates[:, 2 * H:3 * H])
        og = jax.nn.sigmoid(gates[:, 3 * H:4 * H])
        cc = fg * cc + ig * gg
        h = og * jnp.tanh(cc)
    out_ref[...] = jnp.dot(h, wfc_ref[...],
                           preferred_element_type=jnp.float32) + bfc_ref[0]


def _lstm_head(V, Hs, deg, W_ihT, W_hhT, b_ih2, b_hh2, b_g2, W_fcT, b_fc2):
    return pl.pallas_call(
        _lstm_body,
        grid=(N // BN_D,),
        in_specs=[
            pl.BlockSpec((T, BN_D, H), lambda n: (0, n, 0)),
            pl.BlockSpec((T, BN_D, H), lambda n: (0, n, 0)),
            pl.BlockSpec((BN_D, H), lambda n: (n, 0)),
            pl.BlockSpec((H, 4 * H), lambda n: (0, 0)),
            pl.BlockSpec((H, 4 * H), lambda n: (0, 0)),
            pl.BlockSpec((1, 4 * H), lambda n: (0, 0)),
            pl.BlockSpec((1, 4 * H), lambda n: (0, 0)),
            pl.BlockSpec((1, H), lambda n: (0, 0)),
            pl.BlockSpec((H, 3), lambda n: (0, 0)),
            pl.BlockSpec((1, 3), lambda n: (0, 0)),
        ],
        out_specs=pl.BlockSpec((BN_D, 3), lambda n: (n, 0)),
        out_shape=jax.ShapeDtypeStruct((N, 3), jnp.float32),
    )(V, Hs, deg, W_ihT, W_hhT, b_ih2, b_hh2, b_g2, W_fcT, b_fc2)


def kernel(X_seq, edge_index, W_g, b_g, W_ih, W_hh, b_ih, b_hh, W_fc, b_fc):
    if X_seq.ndim == 2:
        X_seq = X_seq[None]
    src = edge_index[0]
    dst = edge_index[1]

    dstpA = jnp.pad(dst.reshape(NS, EPT), ((0, 0), (0, EPAD - EPT)),
                    constant_values=N)
    dstpA = dstpA.reshape(NS, CH, K)

    ones_h = jnp.ones((K, H), jnp.float32)
    zer_h = jnp.zeros((RSZ2, H), jnp.float32)
    deg = _deg_kernel(dstpA, ones_h, zer_h)

    Hs = _gcn_mm(X_seq, W_g, deg)
    Hs_flat = Hs.reshape(T * N, H)

    srcp = jnp.pad(src.reshape(NS, EPT2), ((0, 0), (0, EPAD2 - EPT2)))
    srcp = srcp.reshape(NS, NHLV, CH2, K2)
    dstp = jnp.pad(dst.reshape(NS, EPT2),
                   ((0, 0), (0, EPAD2 - EPT2)), constant_values=N)
    dstp = dstp.reshape(NS, NHLV, CH2, K2)

    Vflat = _agg_kernel(Hs_flat, srcp, dstp)
    V = Vflat.reshape(T, N, H)

    out = _lstm_head(
        V, Hs, deg,
        W_ih.T, W_hh.T,
        b_ih.reshape(1, 4 * H), b_hh.reshape(1, 4 * H),
        b_g.reshape(1, H),
        W_fc.T, b_fc.reshape(1, 3))
    return out


# speedup vs baseline: 1.1040x; 1.1040x over previous
import jax
import jax.numpy as jnp
from jax import lax
from jax.experimental import pallas as pl
from jax.experimental.pallas import tpu as pltpu
from jax.experimental.pallas import tpu_sc as plsc

T, N, F, H, E = 8, 10000, 256, 128, 160000
NC, NS = 2, 16
K = 128
EPT = E // NS
EPAD = 10240
CH = EPAD // K
HALF = N // NC
ROFF2 = 312
RSZ2 = 320
TRB = 5008
NTR = 128
SROWS = TRB + NTR

_mesh = plsc.VectorSubcoreMesh(core_axis_name="c", subcore_axis_name="s",
                               num_cores=NC, num_subcores=NS)


def _mk_local_dst(c, dst_v, lane):
    def mkldst(i, carry):
        for q in range(K // 16):
            d = dst_v[i, pl.ds(q * 16, 16)] - c * HALF
            owned = (d >= 0) & (d < HALF)
            trash = TRB + ((q * 16 + lane) & (NTR - 1))
            dst_v[i, pl.ds(q * 16, 16)] = jnp.where(owned, d, trash)
        return carry

    lax.fori_loop(0, CH, mkldst, 0)


def _deg_body(dstp, ones_h, zer_h, deg_out, dst_v, ones_v, zv, spm):
    c = lax.axis_index("c")
    s = lax.axis_index("s")
    pltpu.sync_copy(dstp.at[s], dst_v)
    pltpu.sync_copy(ones_h, ones_v)
    pltpu.sync_copy(zer_h, zv)
    lane = lax.iota(jnp.int32, 16)
    _mk_local_dst(c, dst_v, lane)
    pltpu.sync_copy(zv, spm.at[pl.ds(s * ROFF2, RSZ2)])
    plsc.subcore_barrier()

    def chunk(j, carry):
        pltpu.sync_copy(ones_v, spm.at[dst_v.at[j]], add=True)
        return carry

    lax.fori_loop(0, CH, chunk, 0)
    plsc.subcore_barrier()
    pltpu.sync_copy(spm.at[pl.ds(s * ROFF2, RSZ2)], zv)
    pltpu.sync_copy(zv, deg_out.at[pl.ds(c * HALF + s * ROFF2, RSZ2)])


_deg_kernel = pl.kernel(
    _deg_body,
    out_type=jax.ShapeDtypeStruct((N, H), jnp.float32),
    mesh=_mesh,
    scratch_types=[
        pltpu.VMEM((CH, K), jnp.int32),
        pltpu.VMEM((K, H), jnp.float32),
        pltpu.VMEM((RSZ2, H), jnp.float32),
        pltpu.VMEM_SHARED((SROWS, H), jnp.float32),
    ],
)


BN_B = 1000


def _gcn_mm_body(x_ref, w_ref, deg_ref, out_ref):
    dis = lax.rsqrt(deg_ref[:, 0] + 1.0)
    h = jnp.dot(x_ref[0], w_ref[...], preferred_element_type=jnp.float32)
    out_ref[0] = h * dis[:, None]


def _gcn_mm(X_seq, W_g, deg):
    return pl.pallas_call(
        _gcn_mm_body,
        grid=(N // BN_B, T),
        in_specs=[
            pl.BlockSpec((1, BN_B, F), lambda n, t: (t, n, 0)),
            pl.BlockSpec((F, H), lambda n, t: (0, 0)),
            pl.BlockSpec((BN_B, H), lambda n, t: (n, 0)),
        ],
        out_specs=pl.BlockSpec((1, BN_B, H), lambda n, t: (t, n, 0)),
        out_shape=jax.ShapeDtypeStruct((T, N, H), jnp.float32),
    )(X_seq, W_g, deg)


TH = T // NC
EPT2 = E // NS
K2 = 64
EPAD2 = 10240
NHLV = 2
CH2 = EPAD2 // K2 // NHLV
ROFF = 624
RSZ = 640
NB = RSZ // K2
AROWS = N + 8


def _agg_body(hs, srcp, dstp, vout, src_v, dst_v, rows0, rows1,
              spm, semg0, semg1):
    c = lax.axis_index("c")
    s = lax.axis_index("s")
    zv = jnp.zeros((16,), jnp.float32)

    def zrow(i, carry):
        for q in range(H // 16):
            rows0[i, pl.ds(q * 16, 16)] = zv
        return carry

    for t in range(TH):
        lax.fori_loop(0, K2, zrow, 0)
        for p in range(NB):
            pltpu.sync_copy(rows0, spm.at[pl.ds(s * ROFF + p * K2, K2)])
        plsc.subcore_barrier()
        off = c * (TH * N) + t * N
        for half in range(NHLV):
            pltpu.sync_copy(srcp.at[s, half], src_v)
            pltpu.sync_copy(dstp.at[s, half], dst_v)

            def bump(i, carry):
                for q in range(K2 // 16):
                    src_v[i, pl.ds(q * 16, 16)] = (
                        src_v[i, pl.ds(q * 16, 16)] + off)
                return carry

            lax.fori_loop(0, CH2, bump, 0)
            pltpu.async_copy(hs.at[src_v.at[0]], rows0, semg0)
            pltpu.async_copy(hs.at[src_v.at[1]], rows1, semg1)

            def pair(m, carry):
                j0 = 2 * m
                pltpu.make_async_copy(hs.at[src_v.at[j0]], rows0,
                                      semg0).wait()
                pltpu.sync_copy(rows0, spm.at[dst_v.at[j0]], add=True)

                @pl.when(m < CH2 // 2 - 1)
                def _():
                    pltpu.async_copy(hs.at[src_v.at[j0 + 2]], rows0, semg0)

                pltpu.make_async_copy(hs.at[src_v.at[j0 + 1]], rows1,
                                      semg1).wait()
                pltpu.sync_copy(rows1, spm.at[dst_v.at[j0 + 1]], add=True)

                @pl.when(m < CH2 // 2 - 1)
                def _():
                    pltpu.async_copy(hs.at[src_v.at[j0 + 3]], rows1, semg1)

                return carry

            lax.fori_loop(0, CH2 // 2, pair, 0)
        plsc.subcore_barrier()
        base = (c * TH + t) * N
        for p in range(NB):
            buf, sem = (rows0, semg0) if p % 2 == 0 else (rows1, semg1)
            if p >= 2:
                pltpu.make_async_copy(
                    buf,
                    vout.at[pl.ds(base + s * ROFF + (p - 2) * K2, K2)],
                    sem).wait()
            pltpu.sync_copy(spm.at[pl.ds(s * ROFF + p * K2, K2)], buf)
            pltpu.async_copy(
                buf, vout.at[pl.ds(base + s * ROFF + p * K2, K2)], sem)
        for p in (NB - 2, NB - 1):
            buf, sem = (rows0, semg0) if p % 2 == 0 else (rows1, semg1)
            pltpu.make_async_copy(
                buf, vout.at[pl.ds(base + s * ROFF + p * K2, K2)],
                sem).wait()


_agg_kernel = pl.kernel(
    _agg_body,
    out_type=jax.ShapeDtypeStruct((T * N, H), jnp.float32),
    mesh=_mesh,
    scratch_types=[
        pltpu.VMEM((CH2, K2), jnp.int32),
        pltpu.VMEM((CH2, K2), jnp.int32),
        pltpu.VMEM((K2, H), jnp.float32),
        pltpu.VMEM((K2, H), jnp.float32),
        pltpu.VMEM_SHARED((AROWS, H), jnp.float32),
        pltpu.SemaphoreType.DMA,
        pltpu.SemaphoreType.DMA,
    ],
)


BN_D = 1000


def _lstm_body(v_ref, hs_ref, deg_ref, wih_ref, whh_ref, bih_ref, bhh_ref,
               bg_ref, wfc_ref, bfc_ref, out_ref):
    dis = lax.rsqrt(deg_ref[:, 0] + 1.0)[:, None]
    bias = bih_ref[0] + bhh_ref[0]
    bg = bg_ref[0]
    h = jnp.zeros((BN_D, H), jnp.float32)
    cc = jnp.zeros((BN_D, H), jnp.float32)
    for t in range(T):
        v_t = v_ref[t] + hs_ref[t]
        g_in = v_t * dis + bg
        gates = (jnp.dot(g_in, wih_ref[...],
                         preferred_element_type=jnp.float32)
                 + jnp.dot(h, whh_ref[...],
                           preferred_element_type=jnp.float32)
                 + bias)
        ig = jax.nn.sigmoid(gates[:, 0:H])
        fg = jax.nn.sigmoid(gates[:, H:2 * H])
        gg = jnp.tanh(gates[:, 2 * H:3 * H])
        og = jax.nn.sigmoid(gates[:, 3 * H:4 * H])
        cc = fg * cc + ig * gg
        h = og * jnp.tanh(cc)
    out_ref[...] = jnp.dot(h, wfc_ref[...],
                           preferred_element_type=jnp.float32) + bfc_ref[0]


def _lstm_head(V, Hs, deg, W_ihT, W_hhT, b_ih2, b_hh2, b_g2, W_fcT, b_fc2):
    return pl.pallas_call(
        _lstm_body,
        grid=(N // BN_D,),
        in_specs=[
            pl.BlockSpec((T, BN_D, H), lambda n: (0, n, 0)),
            pl.BlockSpec((T, BN_D, H), lambda n: (0, n, 0)),
            pl.BlockSpec((BN_D, H), lambda n: (n, 0)),
            pl.BlockSpec((H, 4 * H), lambda n: (0, 0)),
            pl.BlockSpec((H, 4 * H), lambda n: (0, 0)),
            pl.BlockSpec((1, 4 * H), lambda n: (0, 0)),
            pl.BlockSpec((1, 4 * H), lambda n: (0, 0)),
            pl.BlockSpec((1, H), lambda n: (0, 0)),
            pl.BlockSpec((H, 3), lambda n: (0, 0)),
            pl.BlockSpec((1, 3), lambda n: (0, 0)),
        ],
        out_specs=pl.BlockSpec((BN_D, 3), lambda n: (n, 0)),
        out_shape=jax.ShapeDtypeStruct((N, 3), jnp.float32),
    )(V, Hs, deg, W_ihT, W_hhT, b_ih2, b_hh2, b_g2, W_fcT, b_fc2)


def kernel(X_seq, edge_index, W_g, b_g, W_ih, W_hh, b_ih, b_hh, W_fc, b_fc):
    if X_seq.ndim == 2:
        X_seq = X_seq[None]
    src = edge_index[0]
    dst = edge_index[1]

    dstpA = jnp.pad(dst.reshape(NS, EPT), ((0, 0), (0, EPAD - EPT)),
                    constant_values=N)
    dstpA = dstpA.reshape(NS, CH, K)

    ones_h = jnp.ones((K, H), jnp.float32)
    zer_h = jnp.zeros((RSZ2, H), jnp.float32)
    deg = _deg_kernel(dstpA, ones_h, zer_h)

    Hs = _gcn_mm(X_seq, W_g, deg)
    Hs_flat = Hs.reshape(T * N, H)

    srcp = jnp.pad(src.reshape(NS, EPT2), ((0, 0), (0, EPAD2 - EPT2)))
    srcp = srcp.reshape(NS, NHLV, CH2, K2)
    dstp = jnp.pad(dst.reshape(NS, EPT2),
                   ((0, 0), (0, EPAD2 - EPT2)), constant_values=N)
    dstp = dstp.reshape(NS, NHLV, CH2, K2)

    Vflat = _agg_kernel(Hs_flat, srcp, dstp)
    V = Vflat.reshape(T, N, H)

    out = _lstm_head(
        V, Hs, deg,
        W_ih.T, W_hh.T,
        b_ih.reshape(1, 4 * H), b_hh.reshape(1, 4 * H),
        b_g.reshape(1, H),
        W_fc.T, b_fc.reshape(1, 3))
    return out
